# baseline (speedup 1.0000x reference)
import jax
import jax.numpy as jnp
from jax.experimental import pallas as pl
from jax.experimental.pallas import tpu as pltpu


def _copy_body(x_ref, o_ref):
    o_ref[...] = x_ref[...]


def kernel(x):
    return pl.pallas_call(
        _copy_body,
        grid=(1,),
        in_specs=[pl.BlockSpec((16, 576, 512), lambda i: (i, 0, 0))],
        out_specs=pl.BlockSpec((16, 576, 512), lambda i: (i, 0, 0)),
        out_shape=jax.ShapeDtypeStruct(x.shape, x.dtype),
    )(x)


# speedup vs baseline: 1.1191x; 1.1191x over previous
import jax
import jax.numpy as jnp
from jax.experimental import pallas as pl
from jax.experimental.pallas import tpu as pltpu

_BLOCKS = 3
_ROWS = 16 * 576


def _copy_body(x_ref, o_ref):
    o_ref[...] = x_ref[...]


def kernel(x):
    rows_per_block = _ROWS // _BLOCKS
    out = pl.pallas_call(
        _copy_body,
        grid=(_BLOCKS,),
        in_specs=[pl.BlockSpec((rows_per_block, 512), lambda i: (i, 0))],
        out_specs=pl.BlockSpec((rows_per_block, 512), lambda i: (i, 0)),
        out_shape=jax.ShapeDtypeStruct((_ROWS, 512), x.dtype),
    )(x.reshape(_ROWS, 512))
    return out.reshape(x.shape)


# speedup vs baseline: 1.1475x; 1.0253x over previous
import jax
import jax.numpy as jnp
from jax.experimental import pallas as pl
from jax.experimental.pallas import tpu as pltpu

_CHUNKS = 4
_ROWS = 16 * 576
_CH_ROWS = _ROWS // _CHUNKS


def _copy_body(x_ref, o_ref, buf, in_sem, out_sem):
    ins = []
    for i in range(_CHUNKS):
        c = pltpu.make_async_copy(
            x_ref.at[pl.ds(i * _CH_ROWS, _CH_ROWS), :], buf.at[i], in_sem.at[i]
        )
        c.start()
        ins.append(c)
    outs = []
    for i in range(_CHUNKS):
        ins[i].wait()
        c = pltpu.make_async_copy(
            buf.at[i], o_ref.at[pl.ds(i * _CH_ROWS, _CH_ROWS), :], out_sem.at[i]
        )
        c.start()
        outs.append(c)
    for c in outs:
        c.wait()


def kernel(x):
    out = pl.pallas_call(
        _copy_body,
        in_specs=[pl.BlockSpec(memory_space=pl.ANY)],
        out_specs=pl.BlockSpec(memory_space=pl.ANY),
        out_shape=jax.ShapeDtypeStruct((_ROWS, 512), x.dtype),
        scratch_shapes=[
            pltpu.VMEM((_CHUNKS, _CH_ROWS, 512), x.dtype),
            pltpu.SemaphoreType.DMA((_CHUNKS,)),
            pltpu.SemaphoreType.DMA((_CHUNKS,)),
        ],
    )(x.reshape(_ROWS, 512))
    return out.reshape(x.shape)
